# baseline (speedup 1.0000x reference)
import functools

import jax
import jax.numpy as jnp
from jax import lax
from jax.experimental import pallas as pl
from jax.experimental.pallas import tpu as pltpu
from jax.experimental.pallas import tpu_sc as plsc

NC, NS, L = 2, 16, 16
NW = NC * NS
ROWS = 64 * 32
ROW_LEN = 32768
NIDX = 1024
ROWS_PER_W = ROWS // NW
NQUAD = ROWS_PER_W // 4

_mesh = plsc.VectorSubcoreMesh(
    core_axis_name="c", subcore_axis_name="s", num_cores=NC, num_subcores=NS
)


@functools.partial(
    pl.kernel,
    out_type=jax.ShapeDtypeStruct((ROWS, NIDX), jnp.float32),
    mesh=_mesh,
    compiler_params=pltpu.CompilerParams(needs_layout_passes=False),
    scratch_types=[
        pltpu.VMEM((NIDX,), jnp.int32),
        pltpu.VMEM((NIDX,), jnp.int32),
        pltpu.VMEM((NIDX,), jnp.float32),
        pltpu.VMEM((NIDX,), jnp.float32),
        pltpu.VMEM((NIDX,), jnp.float32),
        pltpu.VMEM((NIDX,), jnp.float32),
        pltpu.SemaphoreType.DMA,
        pltpu.SemaphoreType.DMA,
        pltpu.SemaphoreType.DMA,
        pltpu.SemaphoreType.DMA,
        pltpu.SemaphoreType.DMA,
        pltpu.SemaphoreType.DMA,
        pltpu.SemaphoreType.DMA,
        pltpu.SemaphoreType.DMA,
        pltpu.SemaphoreType.DMA,
        pltpu.SemaphoreType.DMA,
    ],
)
def _sc_gather(t_hbm, i_hbm, o_hbm,
               idx0_v, idx1_v, outa_v, outb_v, outc_v, outd_v,
               isem0, isem1, gsa, gsb, gsc, gsd, osa, osb, osc, osd):
    wid = lax.axis_index("s") * NC + lax.axis_index("c")
    base = wid * ROWS_PER_W
    idxs_v = (idx0_v, idx1_v)
    outs_v = (outa_v, outb_v, outc_v, outd_v)
    isems = (isem0, isem1)
    gsems = (gsa, gsb, gsc, gsd)
    osems = (osa, osb, osc, osd)

    def start_idx(row, q):
        pltpu.async_copy(i_hbm.at[row], idxs_v[q], isems[q])

    def wait_idx(row, q):
        pltpu.make_async_copy(i_hbm.at[row], idxs_v[q], isems[q]).wait()

    def issue_gathers(row, q, o):
        fb = row * ROW_LEN

        def g16(i, _):
            sl = pl.ds(i * L, L)
            iv = idxs_v[q][sl] + fb
            pltpu.async_copy(t_hbm.at[iv], outs_v[o].at[sl], gsems[o])
            return 0

        lax.fori_loop(0, NIDX // L, g16, 0, unroll=4)

    def drain_gathers(o):
        pltpu.make_async_copy(t_hbm.at[pl.ds(0, NIDX)], outs_v[o],
                              gsems[o]).wait()

    def start_out(row, o):
        pltpu.async_copy(outs_v[o], o_hbm.at[row], osems[o])

    def wait_out(o):
        pltpu.make_async_copy(outs_v[o], o_hbm.at[base], osems[o]).wait()

    start_idx(base, 0)
    wait_idx(base, 0)
    start_idx(base + 1, 1)
    issue_gathers(base, 0, 0)

    last = NQUAD - 1

    def quad(g, _):
        for j in range(4):
            r = base + 4 * g + j
            q = j % 2
            o = j
            nxt_q = (j + 1) % 2
            nxt_o = (j + 1) % 4
            if j < 3:
                wait_idx(r + 1, nxt_q)
                @pl.when(g > 0)
                def _():
                    wait_out(nxt_o)
                issue_gathers(r + 1, nxt_q, nxt_o)
            else:
                @pl.when(g < last)
                def _():
                    wait_idx(r + 1, nxt_q)
                    wait_out(nxt_o)
                    issue_gathers(r + 1, nxt_q, nxt_o)
            if j < 2:
                start_idx(r + 2, q)
            else:
                @pl.when(g < last)
                def _():
                    start_idx(r + 2, q)
            drain_gathers(o)
            start_out(r, o)
        return 0

    lax.fori_loop(0, NQUAD, quad, 0)
    for o in range(4):
        wait_out(o)


def kernel(tensor, indices):
    t = tensor.reshape(ROWS * ROW_LEN)
    ix = indices.reshape(ROWS, NIDX)
    out = _sc_gather(t, ix)
    return out.reshape(indices.shape)


# speedup vs baseline: 2.1605x; 2.1605x over previous
import functools

import jax
import jax.numpy as jnp
from jax import lax
from jax.experimental import pallas as pl
from jax.experimental.pallas import tpu as pltpu
from jax.experimental.pallas import tpu_sc as plsc

NC, NS, L = 2, 16, 16
NW = NC * NS
ROWS = 64 * 32
ROW_LEN = 32768
NIDX = 1024
ROWS_PER_W = ROWS // NW
NPAIR = ROWS_PER_W // 2

_mesh = plsc.VectorSubcoreMesh(
    core_axis_name="c", subcore_axis_name="s", num_cores=NC, num_subcores=NS
)


@functools.partial(
    pl.kernel,
    out_type=jax.ShapeDtypeStruct((ROWS, NIDX), jnp.float32),
    mesh=_mesh,
    compiler_params=pltpu.CompilerParams(needs_layout_passes=False),
    scratch_types=[
        pltpu.VMEM((ROW_LEN,), jnp.float32),
        pltpu.VMEM((ROW_LEN,), jnp.float32),
        pltpu.VMEM((NIDX,), jnp.int32),
        pltpu.VMEM((NIDX,), jnp.int32),
        pltpu.VMEM((NIDX,), jnp.float32),
        pltpu.VMEM((NIDX,), jnp.float32),
        pltpu.SemaphoreType.DMA,
        pltpu.SemaphoreType.DMA,
        pltpu.SemaphoreType.DMA,
        pltpu.SemaphoreType.DMA,
    ],
)
def _sc_gather(t_hbm, i_hbm, o_hbm, row0_v, row1_v, idx0_v, idx1_v,
               out0_v, out1_v, rsem0, rsem1, osem0, osem1):
    wid = lax.axis_index("s") * NC + lax.axis_index("c")
    base = wid * ROWS_PER_W
    rows_v = (row0_v, row1_v)
    idxs_v = (idx0_v, idx1_v)
    outs_v = (out0_v, out1_v)
    rsems = (rsem0, rsem1)
    osems = (osem0, osem1)

    def start_in(row, p):
        pltpu.async_copy(t_hbm.at[row], rows_v[p], rsems[p])
        pltpu.async_copy(i_hbm.at[row], idxs_v[p], rsems[p])

    def wait_in(row, p):
        pltpu.make_async_copy(t_hbm.at[row], rows_v[p], rsems[p]).wait()
        pltpu.make_async_copy(i_hbm.at[row], idxs_v[p], rsems[p]).wait()

    def wait_out(p):
        pltpu.make_async_copy(outs_v[p], o_hbm.at[base], osems[p]).wait()

    def do_row(row, p):
        def gather16(i, _):
            iv = idxs_v[p][pl.ds(i * L, L)]
            outs_v[p][pl.ds(i * L, L)] = plsc.load_gather(rows_v[p], [iv])
            return 0

        lax.fori_loop(0, NIDX // L, gather16, 0, unroll=4)
        pltpu.async_copy(outs_v[p], o_hbm.at[row], osems[p])

    start_in(base, 0)

    def pair(g, _):
        r0 = base + 2 * g
        start_in(r0 + 1, 1)
        wait_in(r0, 0)
        @pl.when(g > 0)
        def _():
            wait_out(0)
        do_row(r0, 0)
        @pl.when(g < NPAIR - 1)
        def _():
            start_in(r0 + 2, 0)
        wait_in(r0 + 1, 1)
        @pl.when(g > 0)
        def _():
            wait_out(1)
        do_row(r0 + 1, 1)
        return 0

    lax.fori_loop(0, NPAIR, pair, 0)
    wait_out(0)
    wait_out(1)


def kernel(tensor, indices):
    t = tensor.reshape(ROWS, ROW_LEN)
    ix = indices.reshape(ROWS, NIDX)
    out = _sc_gather(t, ix)
    return out.reshape(indices.shape)
